# Initial kernel scaffold; baseline (speedup 1.0000x reference)
import jax
import jax.numpy as jnp
from jax.experimental import pallas as pl


def kernel(x, codebook):
    raise NotImplementedError("write your pallas kernel here")


# speedup vs baseline: 1.3620x; 1.3620x over previous
import functools

import jax
import jax.numpy as jnp
from jax.experimental import pallas as pl
from jax.experimental.pallas import tpu as pltpu

_K = 1024
_D = 64
_BLK = 2048


def _vq_body(x_ref, cb_ref, xsq_ref, csq_ref, idx_ref, q_ref):
    xb = x_ref[...]
    cb = cb_ref[...]
    mm = jax.lax.dot_general(xb, cb, (((1,), (1,)), ((), ())),
                             preferred_element_type=jnp.float32)
    x_sq = xsq_ref[...]
    c_sq = csq_ref[...]
    d2 = x_sq + c_sq - 2.0 * mm
    dist = jnp.sqrt(jnp.maximum(d2, 0.0))
    m = jnp.min(dist, axis=1, keepdims=True)
    lane = jax.lax.broadcasted_iota(jnp.int32, (_BLK, _K), 1)
    idx = jnp.min(jnp.where(dist == m, lane, _K), axis=1).astype(jnp.int32)
    idx_ref[...] = idx
    oh = (lane == idx[:, None]).astype(jnp.float32)
    q_ref[...] = jax.lax.dot_general(oh, cb, (((1,), (0,)), ((), ())),
                                     preferred_element_type=jnp.float32)


@functools.partial(jax.jit, static_argnames=("interpret",))
def _vq(flat_x, codebook, interpret=False):
    n = flat_x.shape[0]
    grid = (n // _BLK,)
    xsq = jnp.sum(flat_x * flat_x, axis=1, keepdims=True)
    csq = jnp.sum(codebook * codebook, axis=1)[None, :]
    idx, q = pl.pallas_call(
        _vq_body,
        grid=grid,
        in_specs=[
            pl.BlockSpec((_BLK, _D), lambda i: (i, 0)),
            pl.BlockSpec((_K, _D), lambda i: (0, 0)),
            pl.BlockSpec((_BLK, 1), lambda i: (i, 0)),
            pl.BlockSpec((1, _K), lambda i: (0, 0)),
        ],
        out_specs=[
            pl.BlockSpec((_BLK,), lambda i: (i,)),
            pl.BlockSpec((_BLK, _D), lambda i: (i, 0)),
        ],
        out_shape=[
            jax.ShapeDtypeStruct((n,), jnp.int32),
            jax.ShapeDtypeStruct((n, _D), jnp.float32),
        ],
        interpret=interpret,
    )(flat_x, codebook, xsq, csq)
    return idx, q


def kernel(x, codebook):
    input_shape = x.shape
    flat_x = x.reshape(-1, codebook.shape[1])
    idx, q = _vq(flat_x, codebook)
    return idx.reshape(input_shape[:-1]), q.reshape(input_shape)
